# Initial kernel scaffold; baseline (speedup 1.0000x reference)
import jax
import jax.numpy as jnp
from jax.experimental import pallas as pl


def kernel(atom_features, atom_edge_index, atom_edge_attr, atom_edge_sh, res_features, atom_res_batch, agg_edge_attr, agg_edge_sh, res_edge_index, res_edge_attr, res_edge_sh, params):
    raise NotImplementedError("write your pallas kernel here")


# speedup vs baseline: 1.4802x; 1.4802x over previous
import functools

import jax
import jax.numpy as jnp
from jax import lax
from jax.experimental import pallas as pl
from jax.experimental.pallas import tpu as pltpu
from jax.experimental.pallas import tpu_sc as plsc

_NC = 2
_NSUB = 16
_NW = _NC * _NSUB

_HID = 64
_SH = 4


def _pad16(d):
    return (d + 15) // 16 * 16


def _sc_gather(table, idx, ch):
    e_tot = idx.shape[0]
    d = table.shape[1]
    per_w = e_tot // _NW
    nch = per_w // ch
    mesh = plsc.VectorSubcoreMesh(core_axis_name="c", subcore_axis_name="s")

    @functools.partial(
        pl.kernel,
        out_type=jax.ShapeDtypeStruct((e_tot, d), jnp.float32),
        mesh=mesh,
        compiler_params=pltpu.CompilerParams(use_tc_tiling_on_sc=False),
        scratch_types=[
            pltpu.VMEM((ch,), jnp.int32),
            pltpu.VMEM((ch, d), jnp.float32),
            pltpu.SemaphoreType.DMA,
        ],
    )
    def gk(idx_hbm, tab_hbm, out_hbm, idx_v, rows_v, sem):
        wid = lax.axis_index("s") * _NC + lax.axis_index("c")
        base = wid * per_w

        def body(j, carry):
            off = base + j * ch
            pltpu.sync_copy(idx_hbm.at[pl.ds(off, ch)], idx_v)
            pltpu.async_copy(tab_hbm.at[idx_v], rows_v, sem).wait()
            pltpu.sync_copy(rows_v, out_hbm.at[pl.ds(off, ch)])
            return carry

        lax.fori_loop(0, nch, body, 0)

    return gk(idx, table)


def _sc_scatter_add(msg, idx3, n_nodes, ch):
    e_tot = msg.shape[0]
    d = msg.shape[1]
    per_w = e_tot // _NW
    nch = per_w // ch
    rpt = n_nodes // _NSUB
    zeros = jnp.zeros((n_nodes, d), jnp.float32)
    mesh = plsc.VectorSubcoreMesh(core_axis_name="c", subcore_axis_name="s")

    @functools.partial(
        pl.kernel,
        out_type=jax.ShapeDtypeStruct((_NC, n_nodes, d), jnp.float32),
        mesh=mesh,
        compiler_params=pltpu.CompilerParams(use_tc_tiling_on_sc=False),
        scratch_types=[
            pltpu.VMEM((nch, ch), jnp.int32),
            pltpu.VMEM((ch, d), jnp.float32),
            pltpu.VMEM_SHARED((n_nodes, d), jnp.float32),
        ],
    )
    def sk(msg_hbm, idx_hbm, z_hbm, out_hbm, idx_v, msg_v, acc_s):
        cid = lax.axis_index("c")
        sid = lax.axis_index("s")
        wid = sid * _NC + cid
        r0 = sid * rpt
        pltpu.sync_copy(z_hbm.at[pl.ds(r0, rpt)], acc_s.at[pl.ds(r0, rpt)])
        plsc.subcore_barrier()
        pltpu.sync_copy(idx_hbm.at[wid], idx_v)
        base = wid * per_w

        def body(j, carry):
            off = base + j * ch
            pltpu.sync_copy(msg_hbm.at[pl.ds(off, ch)], msg_v)
            pltpu.sync_copy(msg_v, acc_s.at[idx_v.at[j]], add=True)
            return carry

        lax.fori_loop(0, nch, body, 0)
        plsc.subcore_barrier()
        pltpu.sync_copy(acc_s.at[pl.ds(r0, rpt)],
                        out_hbm.at[cid, pl.ds(r0, rpt)])

    return sk(msg, idx3, zeros)


def _tc_edge(xs, sh, ea, w1, b1, w2, b2, wo, be):
    e_tot, dp = xs.shape
    ein = ea.shape[1]
    k4 = w2.shape[1]
    dop = wo.shape[1]
    grid = (e_tot // be,)

    def body(xs_ref, sh_ref, ea_ref, w1_ref, b1_ref, w2_ref, b2_ref, wo_ref,
             out_ref, tpw_ref):
        h = jnp.maximum(ea_ref[...] @ w1_ref[...] + b1_ref[...], 0.0)
        w = h @ w2_ref[...] + b2_ref[...]
        x = xs_ref[...]
        s = sh_ref[...]
        for k in range(_SH):
            tpw_ref[:, k * dp:(k + 1) * dp] = (
                x * w[:, k * dp:(k + 1) * dp] * s[:, k:k + 1])
        out_ref[...] = tpw_ref[...] @ wo_ref[...]

    full = lambda a: pl.BlockSpec(a.shape, lambda i: (0, 0))
    return pl.pallas_call(
        body,
        grid=grid,
        in_specs=[
            pl.BlockSpec((be, dp), lambda i: (i, 0)),
            pl.BlockSpec((be, _SH), lambda i: (i, 0)),
            pl.BlockSpec((be, ein), lambda i: (i, 0)),
            full(w1), full(b1), full(w2), full(b2), full(wo),
        ],
        out_specs=pl.BlockSpec((be, dop), lambda i: (i, 0)),
        out_shape=jax.ShapeDtypeStruct((e_tot, dop), jnp.float32),
        scratch_shapes=[pltpu.VMEM((be, k4), jnp.float32)],
    )(xs, sh, ea, w1, b1, w2, b2, wo)


def _tc_post(p0, p1, deg, px, bn):
    n, d = p0.shape
    grid = (n // bn,)

    def body(p0_ref, p1_ref, deg_ref, px_ref, out_ref):
        s = p0_ref[...] + p1_ref[...]
        out_ref[...] = s / jnp.maximum(deg_ref[...], 1.0) + px_ref[...]

    return pl.pallas_call(
        body,
        grid=grid,
        in_specs=[
            pl.BlockSpec((bn, d), lambda i: (i, 0)),
            pl.BlockSpec((bn, d), lambda i: (i, 0)),
            pl.BlockSpec((bn, 1), lambda i: (i, 0)),
            pl.BlockSpec((bn, d), lambda i: (i, 0)),
        ],
        out_specs=pl.BlockSpec((bn, d), lambda i: (i, 0)),
        out_shape=jax.ShapeDtypeStruct((n, d), jnp.float32),
    )(p0, p1, deg, px)


def _tc_post_mm(p0, p1, cnt, rx, wdst):
    n, d = p0.shape

    def body(p0_ref, p1_ref, cnt_ref, rx_ref, wd_ref, out_ref):
        s = p0_ref[...] + p1_ref[...]
        out_ref[...] = (s / jnp.maximum(cnt_ref[...], 1.0)
                        + rx_ref[...] @ wd_ref[...])

    full2 = lambda a: pl.BlockSpec(a.shape, lambda: (0, 0))
    return pl.pallas_call(
        body,
        in_specs=[full2(p0), full2(p1), full2(cnt), full2(rx), full2(wdst)],
        out_specs=full2(p0),
        out_shape=jax.ShapeDtypeStruct((n, d), jnp.float32),
    )(p0, p1, cnt, rx, wdst)


def _prep(p, din, dout):
    dp, dop = _pad16(din), _pad16(dout)
    w1 = p['W1']
    b1 = p['b1'].reshape(1, _HID)
    w2 = p['W2'].reshape(_HID, din, _SH).transpose(0, 2, 1)
    w2 = jnp.pad(w2, ((0, 0), (0, 0), (0, dp - din))).reshape(_HID, _SH * dp)
    b2 = p['b2'].reshape(din, _SH).T
    b2 = jnp.pad(b2, ((0, 0), (0, dp - din))).reshape(1, _SH * dp)
    wo = p['Wout'].reshape(din, _SH, dout).transpose(1, 0, 2)
    wo = jnp.pad(wo, ((0, 0), (0, dp - din), (0, dop - dout)))
    wo = wo.reshape(_SH * dp, dop)
    return w1, b1, w2, b2, wo


def _pad_rows(a, n):
    return jnp.pad(a, ((0, n - a.shape[0]),) + ((0, 0),) * (a.ndim - 1))


def kernel(atom_features, atom_edge_index, atom_edge_attr, atom_edge_sh,
           res_features, atom_res_batch, agg_edge_attr, agg_edge_sh,
           res_edge_index, res_edge_attr, res_edge_sh, params):
    n_atom = atom_features.shape[0]
    n_res = res_features.shape[0]
    e_atom = atom_edge_index.shape[1]
    e_res = res_edge_index.shape[1]
    a_dims = [atom_features.shape[1]] + [p['atom']['Wout'].shape[1]
                                         for p in params]
    r_dims = [res_features.shape[1]] + [p['agg']['Wout'].shape[1]
                                        for p in params]

    na = (n_atom + _NW * 64 - 1) // (_NW * 64) * (_NW * 64)
    nr = (n_res + _NW * 4 - 1) // (_NW * 4) * (_NW * 4)
    ea_pad = (e_atom + _NW * 128 - 1) // (_NW * 128) * (_NW * 128)
    er_pad = (e_res + _NW * 128 - 1) // (_NW * 128) * (_NW * 128)

    ax = _pad_rows(atom_features, na)
    rx = _pad_rows(res_features, nr)

    asrc = _pad_rows(atom_edge_index[1], ea_pad)
    adst3 = _pad_rows(atom_edge_index[0], ea_pad).reshape(_NW, -1, 128)
    a_ea = _pad_rows(atom_edge_attr, ea_pad)
    a_sh = _pad_rows(atom_edge_sh, ea_pad)
    g_ea = _pad_rows(agg_edge_attr, na)
    g_sh = _pad_rows(agg_edge_sh, na)
    arb3 = _pad_rows(atom_res_batch, na).reshape(_NW, -1, 64)
    rsrc = _pad_rows(res_edge_index[1], er_pad)
    rdst3 = _pad_rows(res_edge_index[0], er_pad).reshape(_NW, -1, 128)
    r_ea = _pad_rows(res_edge_attr, er_pad)
    r_sh = _pad_rows(res_edge_sh, er_pad)

    ones_a = _pad_rows(jnp.ones((e_atom, 16), jnp.float32), ea_pad)
    ones_g = _pad_rows(jnp.ones((n_atom, 16), jnp.float32), na)
    ones_r = _pad_rows(jnp.ones((e_res, 16), jnp.float32), er_pad)
    dA = _sc_scatter_add(ones_a, adst3, na, 128)
    dG = _sc_scatter_add(ones_g, arb3, nr, 64)
    dR = _sc_scatter_add(ones_r, rdst3, nr, 128)
    deg_a = (dA[0, :, 0:1] + dA[1, :, 0:1])
    cnt_g = (dG[0, :, 0:1] + dG[1, :, 0:1])
    deg_r = (dR[0, :, 0:1] + dR[1, :, 0:1])

    for l, p in enumerate(params):
        da, da1 = a_dims[l], a_dims[l + 1]
        dr, dr1 = r_dims[l], r_dims[l + 1]
        dpa, dpa1 = _pad16(da), _pad16(da1)
        dpr, dpr1 = _pad16(dr), _pad16(dr1)

        w1, b1, w2, b2, wo = _prep(p['atom'], da, da1)
        xs = _sc_gather(ax, asrc, 128)
        msg = _tc_edge(xs, a_sh, a_ea, w1, b1, w2, b2, wo, 1024)
        pa = _sc_scatter_add(msg, adst3, na, 128)
        ax = _tc_post(pa[0], pa[1], deg_a,
                      jnp.pad(ax, ((0, 0), (0, dpa1 - dpa))), 1024)

        w1, b1, w2, b2, wo = _prep(p['agg'], da1, dr1)
        msg = _tc_edge(ax, g_sh, g_ea, w1, b1, w2, b2, wo, 1024)
        qa = _sc_scatter_add(msg, arb3, nr, 64)
        wdst = jnp.pad(p['Wdst'], ((0, dpr - dr), (0, dpr1 - dr1)))
        rx = _tc_post_mm(qa[0], qa[1], cnt_g, rx, wdst)

        w1, b1, w2, b2, wo = _prep(p['res'], dr1, dr1)
        rs = _sc_gather(rx, rsrc, 128)
        msg = _tc_edge(rs, r_sh, r_ea, w1, b1, w2, b2, wo, 1024)
        pr = _sc_scatter_add(msg, rdst3, nr, 128)
        rx = _tc_post(pr[0], pr[1], deg_r, rx, 1280)

    return ax[:n_atom, :a_dims[-1]], rx[:n_res, :r_dims[-1]]


# speedup vs baseline: 1.5537x; 1.0496x over previous
import functools

import jax
import jax.numpy as jnp
from jax import lax
from jax.experimental import pallas as pl
from jax.experimental.pallas import tpu as pltpu
from jax.experimental.pallas import tpu_sc as plsc

_NC = 2
_NSUB = 16
_NW = _NC * _NSUB

_HID = 64
_SH = 4


def _pad16(d):
    return (d + 15) // 16 * 16


def _sc_gather(table, idx, ch, sup):
    e_tot = idx.shape[0]
    d = table.shape[1]
    per_w = e_tot // _NW
    nch = per_w // ch
    nsup = nch // sup
    half = nsup // 2
    rows = sup * ch
    assert nsup * sup == nch and half * 2 == nsup
    mesh = plsc.VectorSubcoreMesh(core_axis_name="c", subcore_axis_name="s")

    @functools.partial(
        pl.kernel,
        out_type=jax.ShapeDtypeStruct((e_tot, d), jnp.float32),
        mesh=mesh,
        compiler_params=pltpu.CompilerParams(use_tc_tiling_on_sc=False),
        scratch_types=[
            pltpu.VMEM((per_w,), jnp.int32),
            pltpu.VMEM((rows, d), jnp.float32),
            pltpu.VMEM((rows, d), jnp.float32),
            pltpu.SemaphoreType.DMA,
            pltpu.SemaphoreType.DMA,
            pltpu.SemaphoreType.DMA,
            pltpu.SemaphoreType.DMA,
        ],
    )
    def gk(idx_hbm, tab_hbm, out_hbm, idx_v, bufa, bufb, sga, sgb, swa, swb):
        wid = lax.axis_index("s") * _NC + lax.axis_index("c")
        base = wid * per_w
        pltpu.sync_copy(idx_hbm.at[pl.ds(base, per_w)], idx_v)

        def fire_g(s, buf, sem):
            for c in range(sup):
                o = s * rows + c * ch
                pltpu.async_copy(tab_hbm.at[idx_v.at[pl.ds(o, ch)]],
                                 buf.at[pl.ds(c * ch, ch)], sem)

        def wait_g(buf, sem):
            for c in range(sup):
                pltpu.make_async_copy(tab_hbm.at[idx_v.at[pl.ds(0, ch)]],
                                      buf.at[pl.ds(c * ch, ch)], sem).wait()

        def fire_w(s, buf, sem):
            pltpu.async_copy(buf, out_hbm.at[pl.ds(base + s * rows, rows)],
                             sem)

        def wait_w(buf, sem):
            pltpu.make_async_copy(buf, out_hbm.at[pl.ds(base, rows)],
                                  sem).wait()

        fire_g(0, bufa, sga)

        def body(t, carry):
            s0 = 2 * t
            s1 = s0 + 1

            @pl.when(t > 0)
            def _():
                wait_w(bufb, swb)

            fire_g(s1, bufb, sgb)
            wait_g(bufa, sga)
            fire_w(s0, bufa, swa)
            wait_g(bufb, sgb)

            @pl.when(t < half - 1)
            def _():
                wait_w(bufa, swa)
                fire_g(s0 + 2, bufa, sga)

            fire_w(s1, bufb, swb)
            return carry

        lax.fori_loop(0, half, body, 0)
        wait_w(bufa, swa)
        wait_w(bufb, swb)

    return gk(idx, table)


def _sc_scatter_add(msg, idx3, n_nodes, ch, sup):
    e_tot = msg.shape[0]
    d = msg.shape[1]
    per_w = e_tot // _NW
    nch = per_w // ch
    nsup = nch // sup
    half = nsup // 2
    rows = sup * ch
    assert nsup * sup == nch and half * 2 == nsup
    rpt = n_nodes // _NSUB
    zeros = jnp.zeros((n_nodes, d), jnp.float32)
    mesh = plsc.VectorSubcoreMesh(core_axis_name="c", subcore_axis_name="s")

    @functools.partial(
        pl.kernel,
        out_type=jax.ShapeDtypeStruct((_NC, n_nodes, d), jnp.float32),
        mesh=mesh,
        compiler_params=pltpu.CompilerParams(use_tc_tiling_on_sc=False),
        scratch_types=[
            pltpu.VMEM((nch, ch), jnp.int32),
            pltpu.VMEM((rows, d), jnp.float32),
            pltpu.VMEM((rows, d), jnp.float32),
            pltpu.VMEM_SHARED((n_nodes, d), jnp.float32),
            pltpu.SemaphoreType.DMA,
            pltpu.SemaphoreType.DMA,
            pltpu.SemaphoreType.DMA,
            pltpu.SemaphoreType.DMA,
        ],
    )
    def sk(msg_hbm, idx_hbm, z_hbm, out_hbm, idx_v, bufa, bufb, acc_s,
           sla, slb, ssa, ssb):
        cid = lax.axis_index("c")
        sid = lax.axis_index("s")
        wid = sid * _NC + cid
        r0 = sid * rpt
        pltpu.sync_copy(z_hbm.at[pl.ds(r0, rpt)], acc_s.at[pl.ds(r0, rpt)])
        pltpu.sync_copy(idx_hbm.at[wid], idx_v)
        plsc.subcore_barrier()
        base = wid * per_w

        def fire_l(s, buf, sem):
            pltpu.async_copy(msg_hbm.at[pl.ds(base + s * rows, rows)], buf,
                             sem)

        def wait_l(buf, sem):
            pltpu.make_async_copy(msg_hbm.at[pl.ds(base, rows)], buf,
                                  sem).wait()

        def fire_s(s, buf, sem):
            for c in range(sup):
                pltpu.async_copy(buf.at[pl.ds(c * ch, ch)],
                                 acc_s.at[idx_v.at[s * sup + c]], sem,
                                 add=True)

        def wait_s(buf, sem):
            for c in range(sup):
                pltpu.make_async_copy(buf.at[pl.ds(c * ch, ch)],
                                      acc_s.at[idx_v.at[0]], sem).wait()

        fire_l(0, bufa, sla)

        def body(t, carry):
            s0 = 2 * t
            s1 = s0 + 1

            @pl.when(t > 0)
            def _():
                wait_s(bufb, ssb)

            fire_l(s1, bufb, slb)
            wait_l(bufa, sla)
            fire_s(s0, bufa, ssa)
            wait_l(bufb, slb)

            @pl.when(t < half - 1)
            def _():
                wait_s(bufa, ssa)
                fire_l(s0 + 2, bufa, sla)

            fire_s(s1, bufb, ssb)
            return carry

        lax.fori_loop(0, half, body, 0)
        wait_s(bufa, ssa)
        wait_s(bufb, ssb)
        plsc.subcore_barrier()
        pltpu.sync_copy(acc_s.at[pl.ds(r0, rpt)],
                        out_hbm.at[cid, pl.ds(r0, rpt)])

    return sk(msg, idx3, zeros)


def _tc_edge(xs, sh, ea, w1, b1, w2, b2, wo, be):
    e_tot, dp = xs.shape
    ein = ea.shape[1]
    k4 = w2.shape[1]
    dop = wo.shape[1]
    grid = (e_tot // be,)

    def body(xs_ref, sh_ref, ea_ref, w1_ref, b1_ref, w2_ref, b2_ref, wo_ref,
             out_ref, tpw_ref):
        h = jnp.maximum(ea_ref[...] @ w1_ref[...] + b1_ref[...], 0.0)
        w = h @ w2_ref[...] + b2_ref[...]
        x = xs_ref[...]
        s = sh_ref[...]
        for k in range(_SH):
            tpw_ref[:, k * dp:(k + 1) * dp] = (
                x * w[:, k * dp:(k + 1) * dp] * s[:, k:k + 1])
        out_ref[...] = tpw_ref[...] @ wo_ref[...]

    full = lambda a: pl.BlockSpec(a.shape, lambda i: (0, 0))
    return pl.pallas_call(
        body,
        grid=grid,
        in_specs=[
            pl.BlockSpec((be, dp), lambda i: (i, 0)),
            pl.BlockSpec((be, _SH), lambda i: (i, 0)),
            pl.BlockSpec((be, ein), lambda i: (i, 0)),
            full(w1), full(b1), full(w2), full(b2), full(wo),
        ],
        out_specs=pl.BlockSpec((be, dop), lambda i: (i, 0)),
        out_shape=jax.ShapeDtypeStruct((e_tot, dop), jnp.float32),
        scratch_shapes=[pltpu.VMEM((be, k4), jnp.float32)],
    )(xs, sh, ea, w1, b1, w2, b2, wo)


def _tc_post(p0, p1, deg, px, bn):
    n, d = p0.shape
    grid = (n // bn,)

    def body(p0_ref, p1_ref, deg_ref, px_ref, out_ref):
        s = p0_ref[...] + p1_ref[...]
        out_ref[...] = s / jnp.maximum(deg_ref[...], 1.0) + px_ref[...]

    return pl.pallas_call(
        body,
        grid=grid,
        in_specs=[
            pl.BlockSpec((bn, d), lambda i: (i, 0)),
            pl.BlockSpec((bn, d), lambda i: (i, 0)),
            pl.BlockSpec((bn, 1), lambda i: (i, 0)),
            pl.BlockSpec((bn, d), lambda i: (i, 0)),
        ],
        out_specs=pl.BlockSpec((bn, d), lambda i: (i, 0)),
        out_shape=jax.ShapeDtypeStruct((n, d), jnp.float32),
    )(p0, p1, deg, px)


def _tc_post_mm(p0, p1, cnt, rx, wdst):
    n, d = p0.shape

    def body(p0_ref, p1_ref, cnt_ref, rx_ref, wd_ref, out_ref):
        s = p0_ref[...] + p1_ref[...]
        out_ref[...] = (s / jnp.maximum(cnt_ref[...], 1.0)
                        + rx_ref[...] @ wd_ref[...])

    full2 = lambda a: pl.BlockSpec(a.shape, lambda: (0, 0))
    return pl.pallas_call(
        body,
        in_specs=[full2(p0), full2(p1), full2(cnt), full2(rx), full2(wdst)],
        out_specs=full2(p0),
        out_shape=jax.ShapeDtypeStruct((n, d), jnp.float32),
    )(p0, p1, cnt, rx, wdst)


def _prep(p, din, dout):
    dp, dop = _pad16(din), _pad16(dout)
    w1 = p['W1']
    b1 = p['b1'].reshape(1, _HID)
    w2 = p['W2'].reshape(_HID, din, _SH).transpose(0, 2, 1)
    w2 = jnp.pad(w2, ((0, 0), (0, 0), (0, dp - din))).reshape(_HID, _SH * dp)
    b2 = p['b2'].reshape(din, _SH).T
    b2 = jnp.pad(b2, ((0, 0), (0, dp - din))).reshape(1, _SH * dp)
    wo = p['Wout'].reshape(din, _SH, dout).transpose(1, 0, 2)
    wo = jnp.pad(wo, ((0, 0), (0, dp - din), (0, dop - dout)))
    wo = wo.reshape(_SH * dp, dop)
    return w1, b1, w2, b2, wo


def _pad_rows(a, n):
    return jnp.pad(a, ((0, n - a.shape[0]),) + ((0, 0),) * (a.ndim - 1))


def kernel(atom_features, atom_edge_index, atom_edge_attr, atom_edge_sh,
           res_features, atom_res_batch, agg_edge_attr, agg_edge_sh,
           res_edge_index, res_edge_attr, res_edge_sh, params):
    n_atom = atom_features.shape[0]
    n_res = res_features.shape[0]
    e_atom = atom_edge_index.shape[1]
    e_res = res_edge_index.shape[1]
    a_dims = [atom_features.shape[1]] + [p['atom']['Wout'].shape[1]
                                         for p in params]
    r_dims = [res_features.shape[1]] + [p['agg']['Wout'].shape[1]
                                        for p in params]

    na = (n_atom + _NW * 64 - 1) // (_NW * 64) * (_NW * 64)
    nr = (n_res + _NW * 4 - 1) // (_NW * 4) * (_NW * 4)
    ea_pad = (e_atom + _NW * 128 - 1) // (_NW * 128) * (_NW * 128)
    er_pad = (e_res + _NW * 128 - 1) // (_NW * 128) * (_NW * 128)

    ax = _pad_rows(atom_features, na)
    rx = _pad_rows(res_features, nr)

    asrc = _pad_rows(atom_edge_index[1], ea_pad)
    adst3 = _pad_rows(atom_edge_index[0], ea_pad).reshape(_NW, -1, 128)
    a_ea = _pad_rows(atom_edge_attr, ea_pad)
    a_sh = _pad_rows(atom_edge_sh, ea_pad)
    g_ea = _pad_rows(agg_edge_attr, na)
    g_sh = _pad_rows(agg_edge_sh, na)
    arb3 = _pad_rows(atom_res_batch, na).reshape(_NW, -1, 32)
    rsrc = _pad_rows(res_edge_index[1], er_pad)
    rdst3 = _pad_rows(res_edge_index[0], er_pad).reshape(_NW, -1, 128)
    r_ea = _pad_rows(res_edge_attr, er_pad)
    r_sh = _pad_rows(res_edge_sh, er_pad)

    ones_a = _pad_rows(jnp.ones((e_atom, 16), jnp.float32), ea_pad)
    ones_g = _pad_rows(jnp.ones((n_atom, 16), jnp.float32), na)
    ones_r = _pad_rows(jnp.ones((e_res, 16), jnp.float32), er_pad)
    dA = _sc_scatter_add(ones_a, adst3, na, 128, 4)
    dG = _sc_scatter_add(ones_g, arb3, nr, 32, 1)
    dR = _sc_scatter_add(ones_r, rdst3, nr, 128, 1)
    deg_a = (dA[0, :, 0:1] + dA[1, :, 0:1])
    cnt_g = (dG[0, :, 0:1] + dG[1, :, 0:1])
    deg_r = (dR[0, :, 0:1] + dR[1, :, 0:1])

    for l, p in enumerate(params):
        da, da1 = a_dims[l], a_dims[l + 1]
        dr, dr1 = r_dims[l], r_dims[l + 1]
        dpa, dpa1 = _pad16(da), _pad16(da1)
        dpr, dpr1 = _pad16(dr), _pad16(dr1)

        w1, b1, w2, b2, wo = _prep(p['atom'], da, da1)
        xs = _sc_gather(ax, asrc, 128, 4)
        msg = _tc_edge(xs, a_sh, a_ea, w1, b1, w2, b2, wo, 1024)
        pa = _sc_scatter_add(msg, adst3, na, 128, 4)
        ax = _tc_post(pa[0], pa[1], deg_a,
                      jnp.pad(ax, ((0, 0), (0, dpa1 - dpa))), 1024)

        w1, b1, w2, b2, wo = _prep(p['agg'], da1, dr1)
        msg = _tc_edge(ax, g_sh, g_ea, w1, b1, w2, b2, wo, 1024)
        qa = _sc_scatter_add(msg, arb3, nr, 32, 1)
        wdst = jnp.pad(p['Wdst'], ((0, dpr - dr), (0, dpr1 - dr1)))
        rx = _tc_post_mm(qa[0], qa[1], cnt_g, rx, wdst)

        w1, b1, w2, b2, wo = _prep(p['res'], dr1, dr1)
        rs = _sc_gather(rx, rsrc, 128, 1)
        msg = _tc_edge(rs, r_sh, r_ea, w1, b1, w2, b2, wo, 1024)
        pr = _sc_scatter_add(msg, rdst3, nr, 128, 1)
        rx = _tc_post(pr[0], pr[1], deg_r, rx, 1280)

    return ax[:n_atom, :a_dims[-1]], rx[:n_res, :r_dims[-1]]


# speedup vs baseline: 1.7903x; 1.1523x over previous
import functools

import jax
import jax.numpy as jnp
from jax import lax
from jax.experimental import pallas as pl
from jax.experimental.pallas import tpu as pltpu
from jax.experimental.pallas import tpu_sc as plsc

_NC = 2
_NSUB = 16
_NW = _NC * _NSUB

_HID = 64
_SH = 4
_D = 128


def _pad16(d):
    return (d + 15) // 16 * 16


def _sc_gather(table, idx, ch, sup):
    e_tot = idx.shape[0]
    d = table.shape[1]
    per_w = e_tot // _NW
    nch = per_w // ch
    nsup = nch // sup
    half = nsup // 2
    rows = sup * ch
    assert nsup * sup == nch and half * 2 == nsup
    mesh = plsc.VectorSubcoreMesh(core_axis_name="c", subcore_axis_name="s")

    @functools.partial(
        pl.kernel,
        out_type=jax.ShapeDtypeStruct((e_tot, d), jnp.float32),
        mesh=mesh,
        scratch_types=[
            pltpu.VMEM((per_w,), jnp.int32),
            pltpu.VMEM((rows, d), jnp.float32),
            pltpu.VMEM((rows, d), jnp.float32),
            pltpu.SemaphoreType.DMA,
            pltpu.SemaphoreType.DMA,
            pltpu.SemaphoreType.DMA,
            pltpu.SemaphoreType.DMA,
        ],
    )
    def gk(idx_hbm, tab_hbm, out_hbm, idx_v, bufa, bufb, sga, sgb, swa, swb):
        wid = lax.axis_index("s") * _NC + lax.axis_index("c")
        base = wid * per_w
        pltpu.sync_copy(idx_hbm.at[pl.ds(base, per_w)], idx_v)

        def fire_g(s, buf, sem):
            for c in range(sup):
                o = s * rows + c * ch
                pltpu.async_copy(tab_hbm.at[idx_v.at[pl.ds(o, ch)]],
                                 buf.at[pl.ds(c * ch, ch)], sem)

        def wait_g(buf, sem):
            for c in range(sup):
                pltpu.make_async_copy(tab_hbm.at[idx_v.at[pl.ds(0, ch)]],
                                      buf.at[pl.ds(c * ch, ch)], sem).wait()

        def fire_w(s, buf, sem):
            pltpu.async_copy(buf, out_hbm.at[pl.ds(base + s * rows, rows)],
                             sem)

        def wait_w(buf, sem):
            pltpu.make_async_copy(buf, out_hbm.at[pl.ds(base, rows)],
                                  sem).wait()

        fire_g(0, bufa, sga)

        def body(t, carry):
            s0 = 2 * t
            s1 = s0 + 1

            @pl.when(t > 0)
            def _():
                wait_w(bufb, swb)

            fire_g(s1, bufb, sgb)
            wait_g(bufa, sga)
            fire_w(s0, bufa, swa)
            wait_g(bufb, sgb)

            @pl.when(t < half - 1)
            def _():
                wait_w(bufa, swa)
                fire_g(s0 + 2, bufa, sga)

            fire_w(s1, bufb, swb)
            return carry

        lax.fori_loop(0, half, body, 0)
        wait_w(bufa, swa)
        wait_w(bufb, swb)

    return gk(idx, table)


def _sc_scatter_add(msg, idx3, n_nodes, ch, sup):
    e_tot = msg.shape[0]
    d = msg.shape[1]
    per_w = e_tot // _NW
    nch = per_w // ch
    nsup = nch // sup
    half = nsup // 2
    rows = sup * ch
    assert nsup * sup == nch and half * 2 == nsup
    rpt = n_nodes // _NSUB
    zeros = jnp.zeros((n_nodes, d), jnp.float32)
    mesh = plsc.VectorSubcoreMesh(core_axis_name="c", subcore_axis_name="s")

    @functools.partial(
        pl.kernel,
        out_type=jax.ShapeDtypeStruct((_NC, n_nodes, d), jnp.float32),
        mesh=mesh,
        scratch_types=[
            pltpu.VMEM((nch, ch), jnp.int32),
            pltpu.VMEM((rows, d), jnp.float32),
            pltpu.VMEM((rows, d), jnp.float32),
            pltpu.VMEM_SHARED((n_nodes, d), jnp.float32),
            pltpu.SemaphoreType.DMA,
            pltpu.SemaphoreType.DMA,
            pltpu.SemaphoreType.DMA,
            pltpu.SemaphoreType.DMA,
        ],
    )
    def sk(msg_hbm, idx_hbm, z_hbm, out_hbm, idx_v, bufa, bufb, acc_s,
           sla, slb, ssa, ssb):
        cid = lax.axis_index("c")
        sid = lax.axis_index("s")
        wid = sid * _NC + cid
        r0 = sid * rpt
        pltpu.sync_copy(z_hbm.at[pl.ds(r0, rpt)], acc_s.at[pl.ds(r0, rpt)])
        pltpu.sync_copy(idx_hbm.at[wid], idx_v)
        plsc.subcore_barrier()
        base = wid * per_w

        def fire_l(s, buf, sem):
            pltpu.async_copy(msg_hbm.at[pl.ds(base + s * rows, rows)], buf,
                             sem)

        def wait_l(buf, sem):
            pltpu.make_async_copy(msg_hbm.at[pl.ds(base, rows)], buf,
                                  sem).wait()

        def fire_s(s, buf, sem):
            for c in range(sup):
                pltpu.async_copy(buf.at[pl.ds(c * ch, ch)],
                                 acc_s.at[idx_v.at[s * sup + c]], sem,
                                 add=True)

        def wait_s(buf, sem):
            for c in range(sup):
                pltpu.make_async_copy(buf.at[pl.ds(c * ch, ch)],
                                      acc_s.at[idx_v.at[0]], sem).wait()

        fire_l(0, bufa, sla)

        def body(t, carry):
            s0 = 2 * t
            s1 = s0 + 1

            @pl.when(t > 0)
            def _():
                wait_s(bufb, ssb)

            fire_l(s1, bufb, slb)
            wait_l(bufa, sla)
            fire_s(s0, bufa, ssa)
            wait_l(bufb, slb)

            @pl.when(t < half - 1)
            def _():
                wait_s(bufa, ssa)
                fire_l(s0 + 2, bufa, sla)

            fire_s(s1, bufb, ssb)
            return carry

        lax.fori_loop(0, half, body, 0)
        wait_s(bufa, ssa)
        wait_s(bufb, ssb)
        plsc.subcore_barrier()
        pltpu.sync_copy(acc_s.at[pl.ds(r0, rpt)],
                        out_hbm.at[cid, pl.ds(r0, rpt)])

    return sk(msg, idx3, zeros)


def _tc_edge(xs, sh, ea, wts, e_real, e_pad, be):
    w1, b1, w2k, b2k, wok = wts
    dp = w2k[0].shape[1]
    ein = ea.shape[1]
    grid = (e_pad // be,)
    lastb = (e_real - 1) // be

    def body(xs_ref, sh_ref, ea_ref, w1_ref, b1_ref, *wrefs):
        w2_refs = wrefs[0:4]
        b2_refs = wrefs[4:8]
        wo_refs = wrefs[8:12]
        out_ref = wrefs[12]
        i = pl.program_id(0)
        h = jnp.maximum(ea_ref[...] @ w1_ref[...] + b1_ref[...], 0.0)
        x = xs_ref[:, :dp]
        s = sh_ref[...]
        acc = jnp.zeros((be, _D), jnp.float32)
        for k in range(_SH):
            wk = h @ w2_refs[k][...] + b2_refs[k][...]
            acc = acc + (x * wk * s[:, k:k + 1]) @ wo_refs[k][...]
        row = i * be + lax.broadcasted_iota(jnp.int32, (be, 1), 0)
        one127 = (lax.broadcasted_iota(jnp.int32, (1, _D), 1)
                  == (_D - 1)).astype(jnp.float32)
        out_ref[...] = jnp.where(row < e_real, acc + one127, 0.0)

    clamp = lambda a: pl.BlockSpec((be, a.shape[1]),
                                   lambda i: (jnp.minimum(i, lastb), 0))
    full = lambda a: pl.BlockSpec(a.shape, lambda i: (0, 0))
    return pl.pallas_call(
        body,
        grid=grid,
        in_specs=([pl.BlockSpec((be, _D), lambda i: (i, 0)),
                   clamp(sh), clamp(ea), full(w1), full(b1)]
                  + [full(w) for w in w2k] + [full(b) for b in b2k]
                  + [full(w) for w in wok]),
        out_specs=pl.BlockSpec((be, _D), lambda i: (i, 0)),
        out_shape=jax.ShapeDtypeStruct((e_pad, _D), jnp.float32),
    )(xs, sh, ea, w1, b1, *w2k, *b2k, *wok)


def _tc_post(p0, p1, px, bn):
    n = p0.shape[0]
    grid = (n // bn,)

    def body(p0_ref, p1_ref, px_ref, out_ref):
        s = p0_ref[...] + p1_ref[...]
        deg = jnp.maximum(s[:, _D - 1:_D], 1.0)
        keep = (lax.broadcasted_iota(jnp.int32, (1, _D), 1)
                < (_D - 1)).astype(jnp.float32)
        out_ref[...] = (s / deg + px_ref[...]) * keep

    spec = pl.BlockSpec((bn, _D), lambda i: (i, 0))
    return pl.pallas_call(
        body,
        grid=grid,
        in_specs=[spec, spec, spec],
        out_specs=spec,
        out_shape=jax.ShapeDtypeStruct((n, _D), jnp.float32),
    )(p0, p1, px)


def _tc_post_mm(p0, p1, rx, wdst):
    n = p0.shape[0]

    def body(p0_ref, p1_ref, rx_ref, wd_ref, out_ref):
        s = p0_ref[...] + p1_ref[...]
        cnt = jnp.maximum(s[:, _D - 1:_D], 1.0)
        keep = (lax.broadcasted_iota(jnp.int32, (1, _D), 1)
                < (_D - 1)).astype(jnp.float32)
        out_ref[...] = (s / cnt + rx_ref[...] @ wd_ref[...]) * keep

    full = lambda a: pl.BlockSpec(a.shape, lambda: (0, 0))
    return pl.pallas_call(
        body,
        in_specs=[full(p0), full(p1), full(rx), full(wdst)],
        out_specs=full(p0),
        out_shape=jax.ShapeDtypeStruct((n, _D), jnp.float32),
    )(p0, p1, rx, wdst)


def _prep(p, din, dout):
    dp = _pad16(din)
    w1 = p['W1']
    b1 = p['b1'].reshape(1, _HID)
    w2 = p['W2'].reshape(_HID, din, _SH)
    b2 = p['b2'].reshape(din, _SH)
    wo = p['Wout'].reshape(din, _SH, dout)
    w2k = [jnp.pad(w2[:, :, k], ((0, 0), (0, dp - din))) for k in range(_SH)]
    b2k = [jnp.pad(b2[:, k].reshape(1, din), ((0, 0), (0, dp - din)))
           for k in range(_SH)]
    wok = [jnp.pad(wo[:, k, :], ((0, dp - din), (0, _D - dout)))
           for k in range(_SH)]
    return w1, b1, w2k, b2k, wok


def _pad_rows(a, n):
    return jnp.pad(a, ((0, n - a.shape[0]),) + ((0, 0),) * (a.ndim - 1))


def kernel(atom_features, atom_edge_index, atom_edge_attr, atom_edge_sh,
           res_features, atom_res_batch, agg_edge_attr, agg_edge_sh,
           res_edge_index, res_edge_attr, res_edge_sh, params):
    n_atom = atom_features.shape[0]
    n_res = res_features.shape[0]
    e_atom = atom_edge_index.shape[1]
    e_res = res_edge_index.shape[1]
    a_dims = [atom_features.shape[1]] + [p['atom']['Wout'].shape[1]
                                         for p in params]
    r_dims = [res_features.shape[1]] + [p['agg']['Wout'].shape[1]
                                        for p in params]

    na = (n_atom + _NW * 64 - 1) // (_NW * 64) * (_NW * 64)
    nr = (n_res + _NW * 4 - 1) // (_NW * 4) * (_NW * 4)
    ea_pad = (e_atom + _NW * 128 - 1) // (_NW * 128) * (_NW * 128)
    er_pad = (e_res + _NW * 128 - 1) // (_NW * 128) * (_NW * 128)

    ax = jnp.pad(atom_features, ((0, na - n_atom), (0, _D - a_dims[0])))
    rx = jnp.pad(res_features, ((0, nr - n_res), (0, _D - r_dims[0])))

    asrc = _pad_rows(atom_edge_index[1], ea_pad)
    adst3 = _pad_rows(atom_edge_index[0], ea_pad).reshape(_NW, -1, 128)
    arb3 = _pad_rows(atom_res_batch, na).reshape(_NW, -1, 32)
    rsrc = _pad_rows(res_edge_index[1], er_pad)
    rdst3 = _pad_rows(res_edge_index[0], er_pad).reshape(_NW, -1, 128)

    for l, p in enumerate(params):
        da1 = a_dims[l + 1]
        dr, dr1 = r_dims[l], r_dims[l + 1]

        wts = _prep(p['atom'], a_dims[l], da1)
        xs = _sc_gather(ax, asrc, 128, 2)
        msg = _tc_edge(xs, atom_edge_sh, atom_edge_attr, wts, e_atom,
                       ea_pad, 1024)
        pa = _sc_scatter_add(msg, adst3, na, 128, 1)
        ax = _tc_post(pa[0], pa[1], ax, 1024)

        wts = _prep(p['agg'], da1, dr1)
        msg = _tc_edge(ax, agg_edge_sh, agg_edge_attr, wts, n_atom, na, 1024)
        qa = _sc_scatter_add(msg, arb3, nr, 32, 1)
        wdst = jnp.pad(p['Wdst'], ((0, _D - dr), (0, _D - dr1)))
        rx = _tc_post_mm(qa[0], qa[1], rx, wdst)

        wts = _prep(p['res'], dr1, dr1)
        rs = _sc_gather(rx, rsrc, 128, 1)
        msg = _tc_edge(rs, res_edge_sh, res_edge_attr, wts, e_res, er_pad,
                       1024)
        pr = _sc_scatter_add(msg, rdst3, nr, 128, 1)
        rx = _tc_post(pr[0], pr[1], rx, 1280)

    return ax[:n_atom, :a_dims[-1]], rx[:n_res, :r_dims[-1]]


# speedup vs baseline: 1.9032x; 1.0630x over previous
import functools

import jax
import jax.numpy as jnp
from jax import lax
from jax.experimental import pallas as pl
from jax.experimental.pallas import tpu as pltpu
from jax.experimental.pallas import tpu_sc as plsc

_NC = 2
_NSUB = 16
_NW = _NC * _NSUB

_HID = 64
_SH = 4
_D = 128


def _pad16(d):
    return (d + 15) // 16 * 16


def _sc_gather(table, idx, ch, nbuf):
    e_tot = idx.shape[0]
    d = table.shape[1]
    per_w = e_tot // _NW
    nch = per_w // ch
    ngrp = nch // nbuf
    assert ngrp * nbuf == nch
    mesh = plsc.VectorSubcoreMesh(core_axis_name="c", subcore_axis_name="s")

    @functools.partial(
        pl.kernel,
        out_type=jax.ShapeDtypeStruct((e_tot, d), jnp.float32),
        mesh=mesh,
        scratch_types=[
            pltpu.VMEM((per_w,), jnp.int32),
            [pltpu.VMEM((ch, d), jnp.float32)] * nbuf,
            [pltpu.SemaphoreType.DMA] * nbuf,
            [pltpu.SemaphoreType.DMA] * nbuf,
        ],
    )
    def gk(idx_hbm, tab_hbm, out_hbm, idx_v, bufs, sgs, sws):
        wid = lax.axis_index("s") * _NC + lax.axis_index("c")
        base = wid * per_w
        pltpu.sync_copy(idx_hbm.at[pl.ds(base, per_w)], idx_v)

        def fire_g(j, b):
            pltpu.async_copy(tab_hbm.at[idx_v.at[pl.ds(j * ch, ch)]],
                             bufs[b], sgs[b])

        def wait_g(b):
            pltpu.make_async_copy(tab_hbm.at[idx_v.at[pl.ds(0, ch)]],
                                  bufs[b], sgs[b]).wait()

        def fire_w(j, b):
            pltpu.async_copy(bufs[b], out_hbm.at[pl.ds(base + j * ch, ch)],
                             sws[b])

        def wait_w(b):
            pltpu.make_async_copy(bufs[b], out_hbm.at[pl.ds(base, ch)],
                                  sws[b]).wait()

        def body(g, carry):
            j0 = g * nbuf
            for b in range(nbuf):
                @pl.when(g > 0)
                def _(b=b):
                    wait_w(b)
                fire_g(j0 + b, b)
            for b in range(nbuf):
                wait_g(b)
                fire_w(j0 + b, b)
            return carry

        lax.fori_loop(0, ngrp, body, 0)
        for b in range(nbuf):
            wait_w(b)

    return gk(idx, table)


def _sc_scatter_add(msg, idx3, n_nodes, ch, sup):
    e_tot = msg.shape[0]
    d = msg.shape[1]
    per_w = e_tot // _NW
    nch = per_w // ch
    nsup = nch // sup
    half = nsup // 2
    rows = sup * ch
    assert nsup * sup == nch and half * 2 == nsup
    rpt = n_nodes // _NSUB
    zeros = jnp.zeros((n_nodes, d), jnp.float32)
    mesh = plsc.VectorSubcoreMesh(core_axis_name="c", subcore_axis_name="s")

    @functools.partial(
        pl.kernel,
        out_type=jax.ShapeDtypeStruct((_NC, n_nodes, d), jnp.float32),
        mesh=mesh,
        scratch_types=[
            pltpu.VMEM((nch, ch), jnp.int32),
            pltpu.VMEM((rows, d), jnp.float32),
            pltpu.VMEM((rows, d), jnp.float32),
            pltpu.VMEM_SHARED((n_nodes, d), jnp.float32),
            pltpu.SemaphoreType.DMA,
            pltpu.SemaphoreType.DMA,
            pltpu.SemaphoreType.DMA,
            pltpu.SemaphoreType.DMA,
        ],
    )
    def sk(msg_hbm, idx_hbm, z_hbm, out_hbm, idx_v, bufa, bufb, acc_s,
           sla, slb, ssa, ssb):
        cid = lax.axis_index("c")
        sid = lax.axis_index("s")
        wid = sid * _NC + cid
        r0 = sid * rpt
        pltpu.sync_copy(z_hbm.at[pl.ds(r0, rpt)], acc_s.at[pl.ds(r0, rpt)])
        pltpu.sync_copy(idx_hbm.at[wid], idx_v)
        plsc.subcore_barrier()
        base = wid * per_w

        def fire_l(s, buf, sem):
            pltpu.async_copy(msg_hbm.at[pl.ds(base + s * rows, rows)], buf,
                             sem)

        def wait_l(buf, sem):
            pltpu.make_async_copy(msg_hbm.at[pl.ds(base, rows)], buf,
                                  sem).wait()

        def fire_s(s, buf, sem):
            for c in range(sup):
                pltpu.async_copy(buf.at[pl.ds(c * ch, ch)],
                                 acc_s.at[idx_v.at[s * sup + c]], sem,
                                 add=True)

        def wait_s(buf, sem):
            for c in range(sup):
                pltpu.make_async_copy(buf.at[pl.ds(c * ch, ch)],
                                      acc_s.at[idx_v.at[0]], sem).wait()

        fire_l(0, bufa, sla)

        def body(t, carry):
            s0 = 2 * t
            s1 = s0 + 1

            @pl.when(t > 0)
            def _():
                wait_s(bufb, ssb)

            fire_l(s1, bufb, slb)
            wait_l(bufa, sla)
            fire_s(s0, bufa, ssa)
            wait_l(bufb, slb)

            @pl.when(t < half - 1)
            def _():
                wait_s(bufa, ssa)
                fire_l(s0 + 2, bufa, sla)

            fire_s(s1, bufb, ssb)
            return carry

        lax.fori_loop(0, half, body, 0)
        wait_s(bufa, ssa)
        wait_s(bufb, ssb)
        plsc.subcore_barrier()
        pltpu.sync_copy(acc_s.at[pl.ds(r0, rpt)],
                        out_hbm.at[cid, pl.ds(r0, rpt)])

    return sk(msg, idx3, zeros)


def _tc_edge(xs, sh, ea, wts, e_real, e_pad, be):
    w1, b1, w2k, b2k, wok = wts
    dp = w2k[0].shape[1]
    ein = ea.shape[1]
    grid = (e_pad // be,)
    lastb = (e_real - 1) // be

    def body(xs_ref, sh_ref, ea_ref, w1_ref, b1_ref, *wrefs):
        w2_refs = wrefs[0:4]
        b2_refs = wrefs[4:8]
        wo_refs = wrefs[8:12]
        out_ref = wrefs[12]
        i = pl.program_id(0)
        h = jnp.maximum(ea_ref[...] @ w1_ref[...] + b1_ref[...], 0.0)
        x = xs_ref[:, :dp]
        s = sh_ref[...]
        acc = jnp.zeros((be, _D), jnp.float32)
        for k in range(_SH):
            wk = h @ w2_refs[k][...] + b2_refs[k][...]
            acc = acc + (x * wk * s[:, k:k + 1]) @ wo_refs[k][...]
        row = i * be + lax.broadcasted_iota(jnp.int32, (be, 1), 0)
        one127 = (lax.broadcasted_iota(jnp.int32, (1, _D), 1)
                  == (_D - 1)).astype(jnp.float32)
        out_ref[...] = jnp.where(row < e_real, acc + one127, 0.0)

    clamp = lambda a: pl.BlockSpec((be, a.shape[1]),
                                   lambda i: (jnp.minimum(i, lastb), 0))
    full = lambda a: pl.BlockSpec(a.shape, lambda i: (0, 0))
    return pl.pallas_call(
        body,
        grid=grid,
        in_specs=([pl.BlockSpec((be, _D), lambda i: (i, 0)),
                   clamp(sh), clamp(ea), full(w1), full(b1)]
                  + [full(w) for w in w2k] + [full(b) for b in b2k]
                  + [full(w) for w in wok]),
        out_specs=pl.BlockSpec((be, _D), lambda i: (i, 0)),
        out_shape=jax.ShapeDtypeStruct((e_pad, _D), jnp.float32),
    )(xs, sh, ea, w1, b1, *w2k, *b2k, *wok)


def _tc_post(p0, p1, px, bn):
    n = p0.shape[0]
    grid = (n // bn,)

    def body(p0_ref, p1_ref, px_ref, out_ref):
        s = p0_ref[...] + p1_ref[...]
        deg = jnp.maximum(s[:, _D - 1:_D], 1.0)
        keep = (lax.broadcasted_iota(jnp.int32, (1, _D), 1)
                < (_D - 1)).astype(jnp.float32)
        out_ref[...] = (s / deg + px_ref[...]) * keep

    spec = pl.BlockSpec((bn, _D), lambda i: (i, 0))
    return pl.pallas_call(
        body,
        grid=grid,
        in_specs=[spec, spec, spec],
        out_specs=spec,
        out_shape=jax.ShapeDtypeStruct((n, _D), jnp.float32),
    )(p0, p1, px)


def _tc_post_mm(p0, p1, rx, wdst):
    n = p0.shape[0]

    def body(p0_ref, p1_ref, rx_ref, wd_ref, out_ref):
        s = p0_ref[...] + p1_ref[...]
        cnt = jnp.maximum(s[:, _D - 1:_D], 1.0)
        keep = (lax.broadcasted_iota(jnp.int32, (1, _D), 1)
                < (_D - 1)).astype(jnp.float32)
        out_ref[...] = (s / cnt + rx_ref[...] @ wd_ref[...]) * keep

    full = lambda a: pl.BlockSpec(a.shape, lambda: (0, 0))
    return pl.pallas_call(
        body,
        in_specs=[full(p0), full(p1), full(rx), full(wdst)],
        out_specs=full(p0),
        out_shape=jax.ShapeDtypeStruct((n, _D), jnp.float32),
    )(p0, p1, rx, wdst)


def _prep(p, din, dout):
    dp = _pad16(din)
    w1 = p['W1']
    b1 = p['b1'].reshape(1, _HID)
    w2 = p['W2'].reshape(_HID, din, _SH)
    b2 = p['b2'].reshape(din, _SH)
    wo = p['Wout'].reshape(din, _SH, dout)
    w2k = [jnp.pad(w2[:, :, k], ((0, 0), (0, dp - din))) for k in range(_SH)]
    b2k = [jnp.pad(b2[:, k].reshape(1, din), ((0, 0), (0, dp - din)))
           for k in range(_SH)]
    wok = [jnp.pad(wo[:, k, :], ((0, dp - din), (0, _D - dout)))
           for k in range(_SH)]
    return w1, b1, w2k, b2k, wok


def _pad_rows(a, n):
    return jnp.pad(a, ((0, n - a.shape[0]),) + ((0, 0),) * (a.ndim - 1))


def kernel(atom_features, atom_edge_index, atom_edge_attr, atom_edge_sh,
           res_features, atom_res_batch, agg_edge_attr, agg_edge_sh,
           res_edge_index, res_edge_attr, res_edge_sh, params):
    n_atom = atom_features.shape[0]
    n_res = res_features.shape[0]
    e_atom = atom_edge_index.shape[1]
    e_res = res_edge_index.shape[1]
    a_dims = [atom_features.shape[1]] + [p['atom']['Wout'].shape[1]
                                         for p in params]
    r_dims = [res_features.shape[1]] + [p['agg']['Wout'].shape[1]
                                        for p in params]

    na = (n_atom + _NW * 64 - 1) // (_NW * 64) * (_NW * 64)
    nr = (n_res + _NW * 4 - 1) // (_NW * 4) * (_NW * 4)
    ea_pad = (e_atom + _NW * 128 - 1) // (_NW * 128) * (_NW * 128)
    er_pad = (e_res + _NW * 128 - 1) // (_NW * 128) * (_NW * 128)

    ax = jnp.pad(atom_features, ((0, na - n_atom), (0, _D - a_dims[0])))
    rx = jnp.pad(res_features, ((0, nr - n_res), (0, _D - r_dims[0])))

    asrc = _pad_rows(atom_edge_index[1], ea_pad)
    adst3 = _pad_rows(atom_edge_index[0], ea_pad).reshape(_NW, -1, 128)
    arb3 = _pad_rows(atom_res_batch, na).reshape(_NW, -1, 32)
    rsrc = _pad_rows(res_edge_index[1], er_pad)
    rdst3 = _pad_rows(res_edge_index[0], er_pad).reshape(_NW, -1, 128)

    for l, p in enumerate(params):
        da1 = a_dims[l + 1]
        dr, dr1 = r_dims[l], r_dims[l + 1]

        wts = _prep(p['atom'], a_dims[l], da1)
        xs = _sc_gather(ax, asrc, 128, 4)
        msg = _tc_edge(xs, atom_edge_sh, atom_edge_attr, wts, e_atom,
                       ea_pad, 2048)
        pa = _sc_scatter_add(msg, adst3, na, 128, 1)
        ax = _tc_post(pa[0], pa[1], ax, 1024)

        wts = _prep(p['agg'], da1, dr1)
        msg = _tc_edge(ax, agg_edge_sh, agg_edge_attr, wts, n_atom, na, 2048)
        qa = _sc_scatter_add(msg, arb3, nr, 32, 1)
        wdst = jnp.pad(p['Wdst'], ((0, _D - dr), (0, _D - dr1)))
        rx = _tc_post_mm(qa[0], qa[1], rx, wdst)

        wts = _prep(p['res'], dr1, dr1)
        rs = _sc_gather(rx, rsrc, 128, 5)
        msg = _tc_edge(rs, res_edge_sh, res_edge_attr, wts, e_res, er_pad,
                       2048)
        pr = _sc_scatter_add(msg, rdst3, nr, 128, 1)
        rx = _tc_post(pr[0], pr[1], rx, 1280)

    return ax[:n_atom, :a_dims[-1]], rx[:n_res, :r_dims[-1]]


# speedup vs baseline: 1.9197x; 1.0087x over previous
import functools

import jax
import jax.numpy as jnp
from jax import lax
from jax.experimental import pallas as pl
from jax.experimental.pallas import tpu as pltpu
from jax.experimental.pallas import tpu_sc as plsc

_NC = 2
_NSUB = 16
_NW = _NC * _NSUB

_HID = 64
_SH = 4
_D = 128


def _pad16(d):
    return (d + 15) // 16 * 16


def _sc_gather(table, idx, ch, nbuf):
    e_tot = idx.shape[0]
    d = table.shape[1]
    per_w = e_tot // _NW
    nch = per_w // ch
    ngrp = nch // nbuf
    assert ngrp * nbuf == nch
    mesh = plsc.VectorSubcoreMesh(core_axis_name="c", subcore_axis_name="s")

    @functools.partial(
        pl.kernel,
        out_type=jax.ShapeDtypeStruct((e_tot, d), jnp.float32),
        mesh=mesh,
        scratch_types=[
            pltpu.VMEM((per_w,), jnp.int32),
            [pltpu.VMEM((ch, d), jnp.float32)] * nbuf,
            [pltpu.SemaphoreType.DMA] * nbuf,
            [pltpu.SemaphoreType.DMA] * nbuf,
        ],
    )
    def gk(idx_hbm, tab_hbm, out_hbm, idx_v, bufs, sgs, sws):
        wid = lax.axis_index("s") * _NC + lax.axis_index("c")
        base = wid * per_w
        pltpu.sync_copy(idx_hbm.at[pl.ds(base, per_w)], idx_v)

        def fire_g(j, b):
            pltpu.async_copy(tab_hbm.at[idx_v.at[pl.ds(j * ch, ch)]],
                             bufs[b], sgs[b])

        def wait_g(b):
            pltpu.make_async_copy(tab_hbm.at[idx_v.at[pl.ds(0, ch)]],
                                  bufs[b], sgs[b]).wait()

        def fire_w(j, b):
            pltpu.async_copy(bufs[b], out_hbm.at[pl.ds(base + j * ch, ch)],
                             sws[b])

        def wait_w(b):
            pltpu.make_async_copy(bufs[b], out_hbm.at[pl.ds(base, ch)],
                                  sws[b]).wait()

        def body(g, carry):
            j0 = g * nbuf
            for b in range(nbuf):
                @pl.when(g > 0)
                def _(b=b):
                    wait_w(b)
                fire_g(j0 + b, b)
            for b in range(nbuf):
                wait_g(b)
                fire_w(j0 + b, b)
            return carry

        lax.fori_loop(0, ngrp, body, 0)
        for b in range(nbuf):
            wait_w(b)

    return gk(idx, table)


def _sc_scatter_add(msg, idx3, n_nodes, ch, sup):
    e_tot = msg.shape[0]
    d = msg.shape[1]
    per_w = e_tot // _NW
    nch = per_w // ch
    nbuf = sup
    ngrp = nch // nbuf
    assert ngrp * nbuf == nch
    rpt = n_nodes // _NSUB
    zeros = jnp.zeros((n_nodes, d), jnp.float32)
    mesh = plsc.VectorSubcoreMesh(core_axis_name="c", subcore_axis_name="s")

    @functools.partial(
        pl.kernel,
        out_type=jax.ShapeDtypeStruct((_NC, n_nodes, d), jnp.float32),
        mesh=mesh,
        scratch_types=[
            pltpu.VMEM((nch, ch), jnp.int32),
            [pltpu.VMEM((ch, d), jnp.float32)] * nbuf,
            pltpu.VMEM_SHARED((n_nodes, d), jnp.float32),
            [pltpu.SemaphoreType.DMA] * nbuf,
            [pltpu.SemaphoreType.DMA] * nbuf,
        ],
    )
    def sk(msg_hbm, idx_hbm, z_hbm, out_hbm, idx_v, bufs, acc_s, sls, sss):
        cid = lax.axis_index("c")
        sid = lax.axis_index("s")
        wid = sid * _NC + cid
        r0 = sid * rpt
        pltpu.sync_copy(z_hbm.at[pl.ds(r0, rpt)], acc_s.at[pl.ds(r0, rpt)])
        pltpu.sync_copy(idx_hbm.at[wid], idx_v)
        plsc.subcore_barrier()
        base = wid * per_w

        def fire_l(j, b):
            pltpu.async_copy(msg_hbm.at[pl.ds(base + j * ch, ch)], bufs[b],
                             sls[b])

        def wait_l(b):
            pltpu.make_async_copy(msg_hbm.at[pl.ds(base, ch)], bufs[b],
                                  sls[b]).wait()

        def fire_s(j, b):
            pltpu.async_copy(bufs[b], acc_s.at[idx_v.at[j]], sss[b],
                             add=True)

        def wait_s(b):
            pltpu.make_async_copy(bufs[b], acc_s.at[idx_v.at[0]],
                                  sss[b]).wait()

        def body(g, carry):
            j0 = g * nbuf
            for b in range(nbuf):
                @pl.when(g > 0)
                def _(b=b):
                    wait_s(b)
                fire_l(j0 + b, b)
            for b in range(nbuf):
                wait_l(b)
                fire_s(j0 + b, b)
            return carry

        lax.fori_loop(0, ngrp, body, 0)
        for b in range(nbuf):
            wait_s(b)
        plsc.subcore_barrier()
        pltpu.sync_copy(acc_s.at[pl.ds(r0, rpt)],
                        out_hbm.at[cid, pl.ds(r0, rpt)])

    return sk(msg, idx3, zeros)


def _tc_edge(xs, sh, ea, wts, e_real, e_pad, be):
    w1, b1, w2k, b2k, wok = wts
    dp = w2k[0].shape[1]
    ein = ea.shape[1]
    grid = (e_pad // be,)
    lastb = (e_real - 1) // be

    def body(xs_ref, sh_ref, ea_ref, w1_ref, b1_ref, *wrefs):
        w2_refs = wrefs[0:4]
        b2_refs = wrefs[4:8]
        wo_refs = wrefs[8:12]
        out_ref = wrefs[12]
        i = pl.program_id(0)
        h = jnp.maximum(ea_ref[...] @ w1_ref[...] + b1_ref[...], 0.0)
        x = xs_ref[:, :dp]
        s = sh_ref[...]
        acc = jnp.zeros((be, _D), jnp.float32)
        for k in range(_SH):
            wk = h @ w2_refs[k][...] + b2_refs[k][...]
            acc = acc + (x * wk * s[:, k:k + 1]) @ wo_refs[k][...]
        row = i * be + lax.broadcasted_iota(jnp.int32, (be, 1), 0)
        one127 = (lax.broadcasted_iota(jnp.int32, (1, _D), 1)
                  == (_D - 1)).astype(jnp.float32)
        out_ref[...] = jnp.where(row < e_real, acc + one127, 0.0)

    clamp = lambda a: pl.BlockSpec((be, a.shape[1]),
                                   lambda i: (jnp.minimum(i, lastb), 0))
    full = lambda a: pl.BlockSpec(a.shape, lambda i: (0, 0))
    return pl.pallas_call(
        body,
        grid=grid,
        in_specs=([pl.BlockSpec((be, _D), lambda i: (i, 0)),
                   clamp(sh), clamp(ea), full(w1), full(b1)]
                  + [full(w) for w in w2k] + [full(b) for b in b2k]
                  + [full(w) for w in wok]),
        out_specs=pl.BlockSpec((be, _D), lambda i: (i, 0)),
        out_shape=jax.ShapeDtypeStruct((e_pad, _D), jnp.float32),
    )(xs, sh, ea, w1, b1, *w2k, *b2k, *wok)


def _tc_post(p0, p1, px, bn):
    n = p0.shape[0]
    grid = (n // bn,)

    def body(p0_ref, p1_ref, px_ref, out_ref):
        s = p0_ref[...] + p1_ref[...]
        deg = jnp.maximum(s[:, _D - 1:_D], 1.0)
        keep = (lax.broadcasted_iota(jnp.int32, (1, _D), 1)
                < (_D - 1)).astype(jnp.float32)
        out_ref[...] = (s / deg + px_ref[...]) * keep

    spec = pl.BlockSpec((bn, _D), lambda i: (i, 0))
    return pl.pallas_call(
        body,
        grid=grid,
        in_specs=[spec, spec, spec],
        out_specs=spec,
        out_shape=jax.ShapeDtypeStruct((n, _D), jnp.float32),
    )(p0, p1, px)


def _tc_post_mm(p0, p1, rx, wdst):
    n = p0.shape[0]

    def body(p0_ref, p1_ref, rx_ref, wd_ref, out_ref):
        s = p0_ref[...] + p1_ref[...]
        cnt = jnp.maximum(s[:, _D - 1:_D], 1.0)
        keep = (lax.broadcasted_iota(jnp.int32, (1, _D), 1)
                < (_D - 1)).astype(jnp.float32)
        out_ref[...] = (s / cnt + rx_ref[...] @ wd_ref[...]) * keep

    full = lambda a: pl.BlockSpec(a.shape, lambda: (0, 0))
    return pl.pallas_call(
        body,
        in_specs=[full(p0), full(p1), full(rx), full(wdst)],
        out_specs=full(p0),
        out_shape=jax.ShapeDtypeStruct((n, _D), jnp.float32),
    )(p0, p1, rx, wdst)


def _prep(p, din, dout):
    dp = _pad16(din)
    w1 = p['W1']
    b1 = p['b1'].reshape(1, _HID)
    w2 = p['W2'].reshape(_HID, din, _SH)
    b2 = p['b2'].reshape(din, _SH)
    wo = p['Wout'].reshape(din, _SH, dout)
    w2k = [jnp.pad(w2[:, :, k], ((0, 0), (0, dp - din))) for k in range(_SH)]
    b2k = [jnp.pad(b2[:, k].reshape(1, din), ((0, 0), (0, dp - din)))
           for k in range(_SH)]
    wok = [jnp.pad(wo[:, k, :], ((0, dp - din), (0, _D - dout)))
           for k in range(_SH)]
    return w1, b1, w2k, b2k, wok


def _pad_rows(a, n):
    return jnp.pad(a, ((0, n - a.shape[0]),) + ((0, 0),) * (a.ndim - 1))


def kernel(atom_features, atom_edge_index, atom_edge_attr, atom_edge_sh,
           res_features, atom_res_batch, agg_edge_attr, agg_edge_sh,
           res_edge_index, res_edge_attr, res_edge_sh, params):
    n_atom = atom_features.shape[0]
    n_res = res_features.shape[0]
    e_atom = atom_edge_index.shape[1]
    e_res = res_edge_index.shape[1]
    a_dims = [atom_features.shape[1]] + [p['atom']['Wout'].shape[1]
                                         for p in params]
    r_dims = [res_features.shape[1]] + [p['agg']['Wout'].shape[1]
                                        for p in params]

    na = (n_atom + _NW * 64 - 1) // (_NW * 64) * (_NW * 64)
    nr = (n_res + _NW * 4 - 1) // (_NW * 4) * (_NW * 4)
    ea_pad = (e_atom + _NW * 128 - 1) // (_NW * 128) * (_NW * 128)
    er_pad = (e_res + _NW * 128 - 1) // (_NW * 128) * (_NW * 128)

    ax = jnp.pad(atom_features, ((0, na - n_atom), (0, _D - a_dims[0])))
    rx = jnp.pad(res_features, ((0, nr - n_res), (0, _D - r_dims[0])))

    asrc = _pad_rows(atom_edge_index[1], ea_pad)
    adst3 = _pad_rows(atom_edge_index[0], ea_pad).reshape(_NW, -1, 64)
    arb3 = _pad_rows(atom_res_batch, na).reshape(_NW, -1, 32)
    rsrc = _pad_rows(res_edge_index[1], er_pad)
    rdst3 = _pad_rows(res_edge_index[0], er_pad).reshape(_NW, -1, 128)

    for l, p in enumerate(params):
        da1 = a_dims[l + 1]
        dr, dr1 = r_dims[l], r_dims[l + 1]

        wts = _prep(p['atom'], a_dims[l], da1)
        xs = _sc_gather(ax, asrc, 128, 4)
        msg = _tc_edge(xs, atom_edge_sh, atom_edge_attr, wts, e_atom,
                       ea_pad, 2048)
        pa = _sc_scatter_add(msg, adst3, na, 64, 4)
        ax = _tc_post(pa[0], pa[1], ax, 1024)

        wts = _prep(p['agg'], da1, dr1)
        msg = _tc_edge(ax, agg_edge_sh, agg_edge_attr, wts, n_atom, na, 2048)
        qa = _sc_scatter_add(msg, arb3, nr, 32, 5)
        wdst = jnp.pad(p['Wdst'], ((0, _D - dr), (0, _D - dr1)))
        rx = _tc_post_mm(qa[0], qa[1], rx, wdst)

        wts = _prep(p['res'], dr1, dr1)
        rs = _sc_gather(rx, rsrc, 128, 5)
        msg = _tc_edge(rs, res_edge_sh, res_edge_attr, wts, e_res, er_pad,
                       2048)
        pr = _sc_scatter_add(msg, rdst3, nr, 128, 5)
        rx = _tc_post(pr[0], pr[1], rx, 1280)

    return ax[:n_atom, :a_dims[-1]], rx[:n_res, :r_dims[-1]]


# speedup vs baseline: 1.9564x; 1.0191x over previous
import functools

import jax
import jax.numpy as jnp
from jax import lax
from jax.experimental import pallas as pl
from jax.experimental.pallas import tpu as pltpu
from jax.experimental.pallas import tpu_sc as plsc

_NC = 2
_NSUB = 16
_NW = _NC * _NSUB

_HID = 64
_SH = 4
_D = 128


def _pad16(d):
    return (d + 15) // 16 * 16


def _sc_gather(table, idx, ch, nbuf):
    e_tot = idx.shape[0]
    d = table.shape[1]
    per_w = e_tot // _NW
    nch = per_w // ch
    ngrp = nch // nbuf
    assert ngrp * nbuf == nch
    mesh = plsc.VectorSubcoreMesh(core_axis_name="c", subcore_axis_name="s")

    @functools.partial(
        pl.kernel,
        out_type=jax.ShapeDtypeStruct((e_tot, d), jnp.float32),
        mesh=mesh,
        scratch_types=[
            pltpu.VMEM((per_w,), jnp.int32),
            [pltpu.VMEM((ch, d), jnp.float32)] * nbuf,
            [pltpu.SemaphoreType.DMA] * nbuf,
            [pltpu.SemaphoreType.DMA] * nbuf,
        ],
    )
    def gk(idx_hbm, tab_hbm, out_hbm, idx_v, bufs, sgs, sws):
        wid = lax.axis_index("s") * _NC + lax.axis_index("c")
        base = wid * per_w
        pltpu.sync_copy(idx_hbm.at[pl.ds(base, per_w)], idx_v)

        def fire_g(j, b):
            pltpu.async_copy(tab_hbm.at[idx_v.at[pl.ds(j * ch, ch)]],
                             bufs[b], sgs[b])

        def wait_g(b):
            pltpu.make_async_copy(tab_hbm.at[idx_v.at[pl.ds(0, ch)]],
                                  bufs[b], sgs[b]).wait()

        def fire_w(j, b):
            pltpu.async_copy(bufs[b], out_hbm.at[pl.ds(base + j * ch, ch)],
                             sws[b])

        def wait_w(b):
            pltpu.make_async_copy(bufs[b], out_hbm.at[pl.ds(base, ch)],
                                  sws[b]).wait()

        def body(g, carry):
            j0 = g * nbuf
            for b in range(nbuf):
                @pl.when(g > 0)
                def _(b=b):
                    wait_w(b)
                fire_g(j0 + b, b)
            for b in range(nbuf):
                wait_g(b)
                fire_w(j0 + b, b)
            return carry

        lax.fori_loop(0, ngrp, body, 0)
        for b in range(nbuf):
            wait_w(b)

    return gk(idx, table)


def _sc_scatter_add(msg, idx3, n_nodes, ch, sup):
    e_tot = msg.shape[0]
    d = msg.shape[1]
    per_w = e_tot // _NW
    nch = per_w // ch
    nbuf = sup
    ngrp = nch // nbuf
    assert ngrp * nbuf == nch
    rpt = n_nodes // _NSUB
    zeros = jnp.zeros((n_nodes, d), jnp.float32)
    mesh = plsc.VectorSubcoreMesh(core_axis_name="c", subcore_axis_name="s")

    @functools.partial(
        pl.kernel,
        out_type=jax.ShapeDtypeStruct((_NC, n_nodes, d), jnp.float32),
        mesh=mesh,
        scratch_types=[
            pltpu.VMEM((nch, ch), jnp.int32),
            [pltpu.VMEM((ch, d), jnp.float32)] * nbuf,
            pltpu.VMEM_SHARED((n_nodes, d), jnp.float32),
            [pltpu.SemaphoreType.DMA] * nbuf,
            [pltpu.SemaphoreType.DMA] * nbuf,
        ],
    )
    def sk(msg_hbm, idx_hbm, z_hbm, out_hbm, idx_v, bufs, acc_s, sls, sss):
        cid = lax.axis_index("c")
        sid = lax.axis_index("s")
        wid = sid * _NC + cid
        r0 = sid * rpt
        pltpu.sync_copy(z_hbm.at[pl.ds(r0, rpt)], acc_s.at[pl.ds(r0, rpt)])
        pltpu.sync_copy(idx_hbm.at[wid], idx_v)
        plsc.subcore_barrier()
        base = wid * per_w

        def fire_l(j, b):
            pltpu.async_copy(msg_hbm.at[pl.ds(base + j * ch, ch)], bufs[b],
                             sls[b])

        def wait_l(b):
            pltpu.make_async_copy(msg_hbm.at[pl.ds(base, ch)], bufs[b],
                                  sls[b]).wait()

        def fire_s(j, b):
            pltpu.async_copy(bufs[b], acc_s.at[idx_v.at[j]], sss[b],
                             add=True)

        def wait_s(b):
            pltpu.make_async_copy(bufs[b], acc_s.at[idx_v.at[0]],
                                  sss[b]).wait()

        def body(g, carry):
            j0 = g * nbuf
            for b in range(nbuf):
                @pl.when(g > 0)
                def _(b=b):
                    wait_s(b)
                fire_l(j0 + b, b)
            for b in range(nbuf):
                wait_l(b)
                fire_s(j0 + b, b)
            return carry

        lax.fori_loop(0, ngrp, body, 0)
        for b in range(nbuf):
            wait_s(b)
        plsc.subcore_barrier()
        pltpu.sync_copy(acc_s.at[pl.ds(r0, rpt)],
                        out_hbm.at[cid, pl.ds(r0, rpt)])

    return sk(msg, idx3, zeros)


def _tc_edge(xs, sh, ea, wts, e_real, e_pad, be):
    w1, b1, w2k, b2k, wok = wts
    dp = w2k[0].shape[1]
    ein = ea.shape[1]
    grid = (e_pad // be,)
    lastb = (e_real - 1) // be

    def body(xs_ref, sh_ref, ea_ref, w1_ref, b1_ref, *wrefs):
        w2_refs = wrefs[0:4]
        b2_refs = wrefs[4:8]
        wo_refs = wrefs[8:12]
        out_ref = wrefs[12]
        i = pl.program_id(0)
        h = jnp.maximum(ea_ref[...] @ w1_ref[...] + b1_ref[...], 0.0)
        x = xs_ref[:, :dp]
        s = sh_ref[...]
        acc = jnp.zeros((be, _D), jnp.float32)
        for k in range(_SH):
            wk = h @ w2_refs[k][...] + b2_refs[k][...]
            acc = acc + (x * wk * s[:, k:k + 1]) @ wo_refs[k][...]
        row = i * be + lax.broadcasted_iota(jnp.int32, (be, 1), 0)
        one127 = (lax.broadcasted_iota(jnp.int32, (1, _D), 1)
                  == (_D - 1)).astype(jnp.float32)
        out_ref[...] = jnp.where(row < e_real, acc + one127, 0.0)

    clamp = lambda a: pl.BlockSpec((be, a.shape[1]),
                                   lambda i: (jnp.minimum(i, lastb), 0))
    full = lambda a: pl.BlockSpec(a.shape, lambda i: (0, 0))
    return pl.pallas_call(
        body,
        grid=grid,
        in_specs=([pl.BlockSpec((be, _D), lambda i: (i, 0)),
                   clamp(sh), clamp(ea), full(w1), full(b1)]
                  + [full(w) for w in w2k] + [full(b) for b in b2k]
                  + [full(w) for w in wok]),
        out_specs=pl.BlockSpec((be, _D), lambda i: (i, 0)),
        out_shape=jax.ShapeDtypeStruct((e_pad, _D), jnp.float32),
    )(xs, sh, ea, w1, b1, *w2k, *b2k, *wok)


def _tc_post(p0, p1, px, bn):
    n = p0.shape[0]
    grid = (n // bn,)

    def body(p0_ref, p1_ref, px_ref, out_ref):
        s = p0_ref[...] + p1_ref[...]
        deg = jnp.maximum(s[:, _D - 1:_D], 1.0)
        keep = (lax.broadcasted_iota(jnp.int32, (1, _D), 1)
                < (_D - 1)).astype(jnp.float32)
        out_ref[...] = (s / deg + px_ref[...]) * keep

    spec = pl.BlockSpec((bn, _D), lambda i: (i, 0))
    return pl.pallas_call(
        body,
        grid=grid,
        in_specs=[spec, spec, spec],
        out_specs=spec,
        out_shape=jax.ShapeDtypeStruct((n, _D), jnp.float32),
    )(p0, p1, px)


def _tc_post_mm(p0, p1, rx, wdst):
    n = p0.shape[0]

    def body(p0_ref, p1_ref, rx_ref, wd_ref, out_ref):
        s = p0_ref[...] + p1_ref[...]
        cnt = jnp.maximum(s[:, _D - 1:_D], 1.0)
        keep = (lax.broadcasted_iota(jnp.int32, (1, _D), 1)
                < (_D - 1)).astype(jnp.float32)
        out_ref[...] = (s / cnt + rx_ref[...] @ wd_ref[...]) * keep

    full = lambda a: pl.BlockSpec(a.shape, lambda: (0, 0))
    return pl.pallas_call(
        body,
        in_specs=[full(p0), full(p1), full(rx), full(wdst)],
        out_specs=full(p0),
        out_shape=jax.ShapeDtypeStruct((n, _D), jnp.float32),
    )(p0, p1, rx, wdst)


def _prep(p, din, dout):
    dp = _pad16(din)
    w1 = p['W1']
    b1 = p['b1'].reshape(1, _HID)
    w2 = p['W2'].reshape(_HID, din, _SH)
    b2 = p['b2'].reshape(din, _SH)
    wo = p['Wout'].reshape(din, _SH, dout)
    w2k = [jnp.pad(w2[:, :, k], ((0, 0), (0, dp - din))) for k in range(_SH)]
    b2k = [jnp.pad(b2[:, k].reshape(1, din), ((0, 0), (0, dp - din)))
           for k in range(_SH)]
    wok = [jnp.pad(wo[:, k, :], ((0, dp - din), (0, _D - dout)))
           for k in range(_SH)]
    return w1, b1, w2k, b2k, wok


def _pad_rows(a, n):
    return jnp.pad(a, ((0, n - a.shape[0]),) + ((0, 0),) * (a.ndim - 1))


def kernel(atom_features, atom_edge_index, atom_edge_attr, atom_edge_sh,
           res_features, atom_res_batch, agg_edge_attr, agg_edge_sh,
           res_edge_index, res_edge_attr, res_edge_sh, params):
    n_atom = atom_features.shape[0]
    n_res = res_features.shape[0]
    e_atom = atom_edge_index.shape[1]
    e_res = res_edge_index.shape[1]
    a_dims = [atom_features.shape[1]] + [p['atom']['Wout'].shape[1]
                                         for p in params]
    r_dims = [res_features.shape[1]] + [p['agg']['Wout'].shape[1]
                                        for p in params]

    na = (n_atom + _NW * 64 - 1) // (_NW * 64) * (_NW * 64)
    nr = (n_res + _NW * 4 - 1) // (_NW * 4) * (_NW * 4)
    ea_pad = (e_atom + _NW * 128 - 1) // (_NW * 128) * (_NW * 128)
    er_pad = (e_res + _NW * 128 - 1) // (_NW * 128) * (_NW * 128)

    ax = jnp.pad(atom_features, ((0, na - n_atom), (0, _D - a_dims[0])))
    rx = jnp.pad(res_features, ((0, nr - n_res), (0, _D - r_dims[0])))

    asrc = _pad_rows(atom_edge_index[1], ea_pad)
    adst3 = _pad_rows(atom_edge_index[0], ea_pad).reshape(_NW, -1, 64)
    arb3 = _pad_rows(atom_res_batch, na).reshape(_NW, -1, 32)
    rsrc = _pad_rows(res_edge_index[1], er_pad)
    rdst3 = _pad_rows(res_edge_index[0], er_pad).reshape(_NW, -1, 128)

    for l, p in enumerate(params):
        da1 = a_dims[l + 1]
        dr, dr1 = r_dims[l], r_dims[l + 1]

        wts = _prep(p['atom'], a_dims[l], da1)
        xs = _sc_gather(ax, asrc, 128, 4)
        msg = _tc_edge(xs, atom_edge_sh, atom_edge_attr, wts, e_atom,
                       ea_pad, 4096)
        pa = _sc_scatter_add(msg, adst3, na, 64, 4)
        ax = _tc_post(pa[0], pa[1], ax, 1024)

        wts = _prep(p['agg'], da1, dr1)
        msg = _tc_edge(ax, agg_edge_sh, agg_edge_attr, wts, n_atom, na, 2048)
        qa = _sc_scatter_add(msg, arb3, nr, 32, 5)
        wdst = jnp.pad(p['Wdst'], ((0, _D - dr), (0, _D - dr1)))
        rx = _tc_post_mm(qa[0], qa[1], rx, wdst)

        wts = _prep(p['res'], dr1, dr1)
        rs = _sc_gather(rx, rsrc, 128, 5)
        msg = _tc_edge(rs, res_edge_sh, res_edge_attr, wts, e_res, er_pad,
                       4096)
        pr = _sc_scatter_add(msg, rdst3, nr, 128, 5)
        rx = _tc_post(pr[0], pr[1], rx, 1280)

    return ax[:n_atom, :a_dims[-1]], rx[:n_res, :r_dims[-1]]
